# baseline (speedup 1.0000x reference)
import functools

import jax
import jax.numpy as jnp
from jax import lax
from jax.experimental import pallas as pl
from jax.experimental.pallas import tpu as pltpu
from jax.experimental.pallas import tpu_sc as plsc

_INFO = plsc.get_sparse_core_info()
_NC, _NS, _L = _INFO.num_cores, _INFO.num_subcores, _INFO.num_lanes
_NC = 1
_NW = _NC * _NS


def _BLOCKS(nch):
    return [2, 2, 2, 1, 1] if nch == 8 else [2] * (nch // 2)


def _make_gather(num_rows, C, W, HW, A):
    assert num_rows % _NW == 0
    bpw = num_rows // _NW
    nch = bpw // _L
    assert bpw % _L == 0 and bpw % 8 == 0

    mesh = plsc.VectorSubcoreMesh(core_axis_name="c", subcore_axis_name="s",
                                  num_cores=1)

    @functools.partial(
        pl.kernel,
        mesh=mesh,
        out_type=jax.ShapeDtypeStruct((num_rows, C), jnp.float32),
        scratch_types=[
            pltpu.VMEM((bpw,), jnp.int32),
            pltpu.VMEM((bpw,), jnp.int32),
            pltpu.VMEM((bpw, C), jnp.float32),
            pltpu.SemaphoreType.DMA,
            [pltpu.SemaphoreType.DMA] * nch,
            pltpu.SemaphoreType.DMA,
        ],
    )
    def gather_kernel(table_hbm, rows_hbm, cols_hbm, out_hbm,
                      rows_v, cols_v, feat_v, sem_p, sems_g, sem_o):
        wid = lax.axis_index("s") * _NC + lax.axis_index("c")
        base = wid * bpw
        cp_r = pltpu.async_copy(rows_hbm.at[pl.ds(base, bpw)], rows_v, sem_p)
        cp_c = pltpu.async_copy(cols_hbm.at[pl.ds(base, bpw)], cols_v, sem_p)
        cp_r.wait()
        cp_c.wait()
        gathers = []
        for j in range(nch):
            tbase = ((base + j * _L) // A) * HW
            r = rows_v[pl.ds(j * _L, _L)]
            c = cols_v[pl.ds(j * _L, _L)]
            idx = tbase + r * W + c
            gathers.append(pltpu.async_copy(
                table_hbm.at[idx], feat_v.at[pl.ds(j * _L, _L)], sems_g[j]))
        outs = []
        done = 0
        for blk in _BLOCKS(nch):
            for k in range(done, done + blk):
                gathers[k].wait()
            outs.append(pltpu.async_copy(
                feat_v.at[pl.ds(done * _L, blk * _L)],
                out_hbm.at[pl.ds(base + done * _L, blk * _L)], sem_o))
            done += blk
        for o in outs:
            o.wait()

    return gather_kernel


def kernel(feature_map, agent_positions, mask):
    B, T, H, W, C = feature_map.shape
    A = agent_positions.shape[2]
    num_rows = B * T * A
    table = feature_map.reshape(B * T * H * W, C)
    pos = agent_positions.reshape(num_rows, 2)
    rows = pos[:, 0].astype(jnp.int32)
    cols = pos[:, 1].astype(jnp.int32)
    fn = _make_gather(num_rows, C, W, H * W, A)
    out = fn(table, rows, cols)
    return out.reshape(B, T, A, C)


# speedup vs baseline: 1.0025x; 1.0025x over previous
import functools

import jax
import jax.numpy as jnp
from jax import lax
from jax.experimental import pallas as pl
from jax.experimental.pallas import tpu as pltpu
from jax.experimental.pallas import tpu_sc as plsc

_INFO = plsc.get_sparse_core_info()
_NC, _NS, _L = _INFO.num_cores, _INFO.num_subcores, _INFO.num_lanes
_NC = 1
_NW = _NC * _NS


def _BLOCKS(nch):
    return [4, 4] if nch == 8 else [2] * (nch // 2)


def _make_gather(num_rows, C, W, HW, A):
    assert num_rows % _NW == 0
    bpw = num_rows // _NW
    nch = bpw // _L
    assert bpw % _L == 0 and bpw % 8 == 0

    mesh = plsc.VectorSubcoreMesh(core_axis_name="c", subcore_axis_name="s",
                                  num_cores=1)

    @functools.partial(
        pl.kernel,
        mesh=mesh,
        out_type=jax.ShapeDtypeStruct((num_rows, C), jnp.float32),
        scratch_types=[
            pltpu.VMEM((bpw,), jnp.int32),
            pltpu.VMEM((bpw,), jnp.int32),
            pltpu.VMEM((bpw, C), jnp.float32),
            pltpu.SemaphoreType.DMA,
            [pltpu.SemaphoreType.DMA] * nch,
            pltpu.SemaphoreType.DMA,
        ],
    )
    def gather_kernel(table_hbm, rows_hbm, cols_hbm, out_hbm,
                      rows_v, cols_v, feat_v, sem_p, sems_g, sem_o):
        wid = lax.axis_index("s") * _NC + lax.axis_index("c")
        base = wid * bpw
        cp_r = pltpu.async_copy(rows_hbm.at[pl.ds(base, bpw)], rows_v, sem_p)
        cp_c = pltpu.async_copy(cols_hbm.at[pl.ds(base, bpw)], cols_v, sem_p)
        cp_r.wait()
        cp_c.wait()
        gathers = []
        for j in range(nch):
            tbase = ((base + j * _L) // A) * HW
            r = rows_v[pl.ds(j * _L, _L)]
            c = cols_v[pl.ds(j * _L, _L)]
            idx = tbase + r * W + c
            gathers.append(pltpu.async_copy(
                table_hbm.at[idx], feat_v.at[pl.ds(j * _L, _L)], sems_g[j]))
        outs = []
        done = 0
        for blk in _BLOCKS(nch):
            for k in range(done, done + blk):
                gathers[k].wait()
            outs.append(pltpu.async_copy(
                feat_v.at[pl.ds(done * _L, blk * _L)],
                out_hbm.at[pl.ds(base + done * _L, blk * _L)], sem_o))
            done += blk
        for o in outs:
            o.wait()

    return gather_kernel


def kernel(feature_map, agent_positions, mask):
    B, T, H, W, C = feature_map.shape
    A = agent_positions.shape[2]
    num_rows = B * T * A
    table = feature_map.reshape(B * T * H * W, C)
    pos = agent_positions.reshape(num_rows, 2)
    rows = pos[:, 0].astype(jnp.int32)
    cols = pos[:, 1].astype(jnp.int32)
    fn = _make_gather(num_rows, C, W, H * W, A)
    out = fn(table, rows, cols)
    return out.reshape(B, T, A, C)


# speedup vs baseline: 1.0078x; 1.0053x over previous
import functools

import jax
import jax.numpy as jnp
from jax import lax
from jax.experimental import pallas as pl
from jax.experimental.pallas import tpu as pltpu
from jax.experimental.pallas import tpu_sc as plsc

_INFO = plsc.get_sparse_core_info()
_NS, _L = _INFO.num_subcores, _INFO.num_lanes
_NC = 1
_NW = _NC * _NS


def _make_gather(num_rows, C, W, HW, A):
    assert num_rows % _NW == 0
    bpw = num_rows // _NW
    nch = bpw // _L
    assert bpw % _L == 0 and bpw % 8 == 0 and A % _L == 0

    mesh = plsc.VectorSubcoreMesh(core_axis_name="c", subcore_axis_name="s",
                                  num_cores=_NC)

    @functools.partial(
        pl.kernel,
        mesh=mesh,
        out_type=jax.ShapeDtypeStruct((num_rows, C), jnp.float32),
        scratch_types=[
            pltpu.VMEM((bpw,), jnp.int32),
            pltpu.VMEM((bpw,), jnp.int32),
            pltpu.VMEM((bpw, C), jnp.float32),
            pltpu.SemaphoreType.DMA,
            [pltpu.SemaphoreType.DMA] * nch,
            pltpu.SemaphoreType.DMA,
        ],
    )
    def gather_kernel(table_hbm, rows_hbm, cols_hbm, out_hbm,
                      rows_v, cols_v, feat_v, sem_p, sems_g, sem_o):
        wid = lax.axis_index("s") * _NC + lax.axis_index("c")
        base = wid * bpw
        cp_r = pltpu.async_copy(rows_hbm.at[pl.ds(base, bpw)], rows_v, sem_p)
        cp_c = pltpu.async_copy(cols_hbm.at[pl.ds(base, bpw)], cols_v, sem_p)
        cp_r.wait()
        cp_c.wait()
        gathers = []
        for j in range(nch):
            tbase = ((base + j * _L) // A) * HW
            r = rows_v[pl.ds(j * _L, _L)]
            c = cols_v[pl.ds(j * _L, _L)]
            idx = tbase + r * W + c
            gathers.append(pltpu.async_copy(
                table_hbm.at[idx], feat_v.at[pl.ds(j * _L, _L)], sems_g[j]))
        outs = []
        for k in range(nch // 2):
            gathers[2 * k].wait()
            gathers[2 * k + 1].wait()
            outs.append(pltpu.async_copy(
                feat_v.at[pl.ds(2 * k * _L, 2 * _L)],
                out_hbm.at[pl.ds(base + 2 * k * _L, 2 * _L)], sem_o))
        for o in outs:
            o.wait()

    return gather_kernel


def kernel(feature_map, agent_positions, mask):
    B, T, H, W, C = feature_map.shape
    A = agent_positions.shape[2]
    num_rows = B * T * A
    table = feature_map.reshape(B * T * H * W, C)
    pos = agent_positions.reshape(num_rows, 2)
    rows = pos[:, 0].astype(jnp.int32)
    cols = pos[:, 1].astype(jnp.int32)
    fn = _make_gather(num_rows, C, W, H * W, A)
    out = fn(table, rows, cols)
    return out.reshape(B, T, A, C)
